# Initial kernel scaffold; baseline (speedup 1.0000x reference)
import functools

import jax
import jax.numpy as jnp
from jax import lax
from jax.experimental import pallas as pl
from jax.experimental.pallas import tpu as pltpu
from jax.experimental.pallas import tpu_sc as plsc

_N = 10000
_D = 128
_E = 320000
_NREL = 200
_NC = 2
_NS = 16
_NW = _NC * _NS
_EPW = _E // _NW
_B = 80
_NCH = _EPW // _B
_RPS = _N // _NS
_ZR = 125

_mesh = plsc.VectorSubcoreMesh(core_axis_name="c", subcore_axis_name="s")


@functools.partial(
    pl.kernel,
    out_type=jax.ShapeDtypeStruct((_NC, _N, _D), jnp.float32),
    mesh=_mesh,
    scratch_types=[
        pltpu.VMEM((_NCH, _B), jnp.int32),
        pltpu.VMEM((_NCH, _B), jnp.int32),
        pltpu.VMEM((_EPW,), jnp.int32),
        pltpu.VMEM((208,), jnp.float32),
        pltpu.VMEM((_B, _D), jnp.float32),
        pltpu.VMEM((_ZR, _D), jnp.float32),
        pltpu.VMEM_SHARED((_N, _D), jnp.float32),
        pltpu.SemaphoreType.DMA,
    ],
)
def _sc_agg(h_hbm, src_hbm, dst_hbm, rel_hbm, alpha_hbm, out_hbm,
            src_v, dst_v, rel_v, alpha_v, rows_v, zbuf_v, agg_sh, sem):
    cid = lax.axis_index("c")
    sid = lax.axis_index("s")
    wid = cid * _NS + sid

    pltpu.sync_copy(src_hbm.at[wid], src_v)
    pltpu.sync_copy(dst_hbm.at[wid], dst_v)
    pltpu.sync_copy(rel_hbm.at[wid], rel_v)
    pltpu.sync_copy(alpha_hbm, alpha_v)

    zv = jnp.zeros((16,), jnp.float32)

    def _zrow(r, carry):
        for c in range(_D // 16):
            zbuf_v[r, pl.ds(c * 16, 16)] = zv
        return carry

    lax.fori_loop(0, _ZR, _zrow, 0)
    for k in range(_RPS // _ZR):
        pltpu.sync_copy(zbuf_v, agg_sh.at[pl.ds(sid * _RPS + k * _ZR, _ZR)])
    plsc.subcore_barrier()

    def _chunk(j, carry):
        pltpu.async_copy(h_hbm.at[src_v.at[j]], rows_v, sem).wait()
        for eb in range(_B // 16):
            rel16 = rel_v[pl.ds(j * _B + eb * 16, 16)]
            a16 = plsc.load_gather(alpha_v, [rel16])
            for e in range(16):
                lane = jnp.full((16,), e, jnp.int32)
                ae = jnp.take(a16, lane,
                              mode=lax.GatherScatterMode.PROMISE_IN_BOUNDS)
                row = eb * 16 + e
                for cc in range(_D // 16):
                    sl = pl.ds(cc * 16, 16)
                    rows_v[row, sl] = rows_v[row, sl] * ae
        pltpu.sync_copy(rows_v, agg_sh.at[dst_v.at[j]], add=True)
        return carry

    lax.fori_loop(0, _NCH, _chunk, 0)
    plsc.subcore_barrier()

    for k in range(_RPS // _ZR):
        sl = pl.ds(sid * _RPS + k * _ZR, _ZR)
        pltpu.sync_copy(agg_sh.at[sl], zbuf_v)
        pltpu.sync_copy(zbuf_v, out_hbm.at[cid].at[sl])


def _tc_body(agg_ref, h_ref, w_ref, b_ref, g_ref, be_ref, out_ref):
    x = agg_ref[0] + agg_ref[1] + h_ref[...]
    y = jnp.dot(x, w_ref[...], preferred_element_type=jnp.float32)
    y = y + b_ref[...]
    mu = jnp.mean(y, axis=0, keepdims=True)
    d = y - mu
    var = jnp.mean(d * d, axis=0, keepdims=True)
    out_ref[...] = jnp.tanh(d * lax.rsqrt(var + 1e-5) * g_ref[...] + be_ref[...])


_tc_layer = pl.pallas_call(
    _tc_body,
    out_shape=jax.ShapeDtypeStruct((_N, _D), jnp.float32),
)


def kernel(entity_embed, edge, alpha0, W0, b0, gamma0, beta0,
           alpha1, W1, b1, gamma1, beta1):
    edge = edge.astype(jnp.int32)
    src = edge[:, 0].reshape(_NW, _NCH, _B)
    dst = edge[:, 2].reshape(_NW, _NCH, _B)
    rel = (edge[:, 1] % _NREL).reshape(_NW, _EPW)
    pad = jnp.zeros((8,), jnp.float32)
    a0 = jnp.concatenate([alpha0, pad])
    a1 = jnp.concatenate([alpha1, pad])
    b0r, g0r, be0r = b0.reshape(1, _D), gamma0.reshape(1, _D), beta0.reshape(1, _D)
    b1r, g1r, be1r = b1.reshape(1, _D), gamma1.reshape(1, _D), beta1.reshape(1, _D)

    agg = _sc_agg(entity_embed, src, dst, rel, a0)
    h1 = _tc_layer(agg, entity_embed, W0, b0r, g0r, be0r)
    agg2 = _sc_agg(h1, src, dst, rel, a1)
    h2 = _tc_layer(agg2, h1, W1, b1r, g1r, be1r)
    return h2


# speedup vs baseline: 3.6496x; 3.6496x over previous
import functools

import jax
import jax.numpy as jnp
from jax import lax
from jax.experimental import pallas as pl
from jax.experimental.pallas import tpu as pltpu
from jax.experimental.pallas import tpu_sc as plsc

_N = 10000
_D = 128
_E = 320000
_NREL = 200
_NC = 2
_NS = 16
_NW = _NC * _NS
_EPW = _E // _NW
_B = 80
_NCH = _EPW // _B
_NPAD = 10240
_RPS = _NPAD // _NS
_ZR = 16

_mesh = plsc.VectorSubcoreMesh(core_axis_name="c", subcore_axis_name="s")

_GDN = lax.GatherDimensionNumbers(
    offset_dims=(), collapsed_slice_dims=(0,), start_index_map=(0,))


def _lane_splat(vec16, lane):
    idx = jnp.full((16, 1), lane, jnp.int32)
    return lax.gather(vec16, idx, _GDN, (1,),
                      mode=lax.GatherScatterMode.PROMISE_IN_BOUNDS)


@functools.partial(
    pl.kernel,
    out_type=jax.ShapeDtypeStruct((_NC, _NPAD, _D), jnp.float32),
    mesh=_mesh,
    scratch_types=[
        pltpu.VMEM((_B,), jnp.int32),
        pltpu.VMEM((_B,), jnp.int32),
        pltpu.VMEM((_B,), jnp.int32),
        pltpu.VMEM((_B,), jnp.float32),
        pltpu.VMEM((_B, _D), jnp.float32),
        pltpu.VMEM((_ZR, _D), jnp.float32),
        pltpu.VMEM_SHARED((_NPAD, _D), jnp.float32),
        pltpu.SemaphoreType.DMA,
        pltpu.SemaphoreType.DMA,
    ],
)
def _sc_agg(h_hbm, src_hbm, dst_hbm, rel_hbm, alpha_hbm, out_hbm,
            src_c, dst_c, rel_c, ach_v, rows_v, zbuf_v, agg_sh, sem, sem2):
    cid = lax.axis_index("c")
    sid = lax.axis_index("s")
    wid = cid * _NS + sid

    zv = jnp.zeros((16,), jnp.float32)
    for r in range(_ZR):
        for c in range(_D // 16):
            zbuf_v[r, pl.ds(c * 16, 16)] = zv

    def _zcp(k, carry):
        pltpu.sync_copy(zbuf_v, agg_sh.at[pl.ds(sid * _RPS + k * _ZR, _ZR)])
        return carry

    lax.fori_loop(0, _RPS // _ZR, _zcp, 0)
    plsc.subcore_barrier()

    def _chunk(j, carry):
        pltpu.sync_copy(src_hbm.at[wid, j], src_c)
        pltpu.sync_copy(dst_hbm.at[wid, j], dst_c)
        pltpu.sync_copy(rel_hbm.at[wid, j], rel_c)
        cp_rows = pltpu.async_copy(h_hbm.at[src_c], rows_v, sem)
        cp_a = pltpu.async_copy(alpha_hbm.at[rel_c], ach_v, sem2)
        cp_a.wait()
        cp_rows.wait()
        for eb in range(_B // 16):
            a16 = ach_v[pl.ds(eb * 16, 16)]
            for e in range(16):
                ae = _lane_splat(a16, e)
                row = eb * 16 + e
                for cc in range(_D // 16):
                    sl = pl.ds(cc * 16, 16)
                    rows_v[row, sl] = rows_v[row, sl] * ae
        pltpu.sync_copy(rows_v, agg_sh.at[dst_c], add=True)
        return carry

    lax.fori_loop(0, _NCH, _chunk, 0)
    plsc.subcore_barrier()

    sl = pl.ds(sid * _RPS, _RPS)
    pltpu.sync_copy(agg_sh.at[sl], out_hbm.at[cid].at[sl])


def _tc_body(agg_ref, h_ref, w_ref, b_ref, g_ref, be_ref, out_ref):
    x = agg_ref[0, :_N] + agg_ref[1, :_N] + h_ref[...]
    y = jnp.dot(x, w_ref[...], preferred_element_type=jnp.float32)
    y = y + b_ref[...]
    mu = jnp.mean(y, axis=0, keepdims=True)
    d = y - mu
    var = jnp.mean(d * d, axis=0, keepdims=True)
    out_ref[...] = jnp.tanh(d * lax.rsqrt(var + 1e-5) * g_ref[...] + be_ref[...])


_tc_layer = pl.pallas_call(
    _tc_body,
    out_shape=jax.ShapeDtypeStruct((_N, _D), jnp.float32),
)


def kernel(entity_embed, edge, alpha0, W0, b0, gamma0, beta0,
           alpha1, W1, b1, gamma1, beta1):
    edge = edge.astype(jnp.int32)
    src = edge[:, 0].reshape(_NW, _NCH, _B)
    dst = edge[:, 2].reshape(_NW, _NCH, _B)
    rel = (edge[:, 1] % _NREL).reshape(_NW, _NCH, _B)
    a0 = alpha0
    a1 = alpha1
    b0r, g0r, be0r = b0.reshape(1, _D), gamma0.reshape(1, _D), beta0.reshape(1, _D)
    b1r, g1r, be1r = b1.reshape(1, _D), gamma1.reshape(1, _D), beta1.reshape(1, _D)

    agg = _sc_agg(entity_embed, src, dst, rel, a0)
    h1 = _tc_layer(agg, entity_embed, W0, b0r, g0r, be0r)
    agg2 = _sc_agg(h1, src, dst, rel, a1)
    h2 = _tc_layer(agg2, h1, W1, b1r, g1r, be1r)
    return h2
